# baseline (speedup 1.0000x reference)
import functools

import jax
import jax.numpy as jnp
from jax import lax
from jax.experimental import pallas as pl
from jax.experimental.pallas import tpu as pltpu
from jax.experimental.pallas import tpu_sc as plsc

_CH = 32
_NCH = 16


def _sc_body(x_hbm, idx_hbm, tok_hbm, out_hbm, buf, rows_v, idx_v,
             sem_in0, sem_in1, sem_out0, sem_out1, sem_tok, *, B, S, D):
    nc = 2
    wid = lax.axis_index("s") * nc + lax.axis_index("c")
    per_w = (B * S) // 32
    wpb = S // per_w
    b = wid // wpb
    r0 = (wid % wpb) * per_w

    sem_in = (sem_in0, sem_in1)
    sem_out = (sem_out0, sem_out1)

    def start_in(i):
        k = i % 2
        return pltpu.async_copy(
            x_hbm.at[b, pl.ds(r0 + i * _CH, _CH), :], buf.at[k], sem_in[k])

    def start_out(i):
        k = i % 2
        return pltpu.async_copy(
            buf.at[k], out_hbm.at[b, pl.ds(1 + r0 + i * _CH, _CH), :],
            sem_out[k])

    pending_out = [None, None]
    cin = start_in(0)
    for i in range(_NCH):
        k = i % 2
        if i + 1 < _NCH:
            if pending_out[(i + 1) % 2] is not None:
                pending_out[(i + 1) % 2].wait()
                pending_out[(i + 1) % 2] = None
            nxt = start_in(i + 1)
        cin.wait()
        pending_out[k] = start_out(i)
        if i + 1 < _NCH:
            cin = nxt
    for p in pending_out:
        if p is not None:
            p.wait()

    @pl.when(wid < B)
    def _():
        pltpu.sync_copy(idx_hbm, idx_v)
        pltpu.async_copy(tok_hbm.at[idx_v], rows_v, sem_tok).wait()
        pltpu.sync_copy(rows_v.at[pl.ds(0, 1)], out_hbm.at[wid, pl.ds(0, 1)])


def kernel(x, modality_id, tokens):
    B, S, D = x.shape
    idx = jnp.full((16,), modality_id, jnp.int32)
    mesh = plsc.VectorSubcoreMesh(core_axis_name="c", subcore_axis_name="s")
    k = pl.kernel(
        functools.partial(_sc_body, B=B, S=S, D=D),
        out_type=jax.ShapeDtypeStruct((B, S + 1, D), x.dtype),
        mesh=mesh,
        compiler_params=pltpu.CompilerParams(use_tc_tiling_on_sc=False),
        scratch_types=[
            pltpu.VMEM((2, _CH, D), x.dtype),
            pltpu.VMEM((16, D), x.dtype),
            pltpu.VMEM((16,), jnp.int32),
            pltpu.SemaphoreType.DMA,
            pltpu.SemaphoreType.DMA,
            pltpu.SemaphoreType.DMA,
            pltpu.SemaphoreType.DMA,
            pltpu.SemaphoreType.DMA,
        ],
    )
    return k(x, idx, tokens)


# speedup vs baseline: 2.1551x; 2.1551x over previous
import functools

import jax
import jax.numpy as jnp
from jax import lax
from jax.experimental import pallas as pl
from jax.experimental.pallas import tpu as pltpu
from jax.experimental.pallas import tpu_sc as plsc

_CH = 32
_W = _CH + 8
_NCH = 16


def _sc_body(x_hbm, idx_hbm, tok_hbm, out_hbm,
             b0, b1, b2, idx_v,
             si0, si1, si2, so0, so1, so2, *, B, S, D):
    nc = 2
    bins = (b0, b1, b2)
    sin = (si0, si1, si2)
    sout = (so0, so1, so2)
    nlane = 16
    nseg = D // nlane

    wid = lax.axis_index("s") * nc + lax.axis_index("c")
    q = wid % 8
    b = wid // 8
    r0 = q * (_CH * _NCH)

    pltpu.sync_copy(idx_hbm, idx_v)
    idxvec = idx_v[...]

    @pl.when(q == 0)
    def _():
        pltpu.sync_copy(tok_hbm, bins[2].at[pl.ds(0, 8)])

    def start_in(i):
        k = i % 3
        ws = pl.multiple_of(jnp.maximum(r0 + i * _CH - 8, 0), 8)
        return pltpu.async_copy(
            x_hbm.at[b, pl.ds(ws, _W), :], bins[k], sin[k])

    def start_out(i):
        k = i % 3
        return pltpu.async_copy(
            bins[k].at[pl.ds(0, _CH)],
            out_hbm.at[b, pl.ds(pl.multiple_of(r0 + i * _CH, 8), _CH), :],
            sout[k])

    def shuf_row(k, dst_r, src_r):
        for c in range(nseg):
            bins[k][dst_r, pl.ds(c * nlane, nlane)] = (
                bins[k][src_r, pl.ds(c * nlane, nlane)])

    def shuffle_asc(k):
        def body(r, _):
            shuf_row(k, r, r + 7)
            return 0
        lax.fori_loop(0, _CH, body, 0)

    def shuffle_desc(k):
        def body(j, _):
            r = _CH - 1 - j
            shuf_row(k, r, r - 1)
            return 0
        lax.fori_loop(0, _CH - 1, body, 0)

    def patch_token(k):
        for c in range(nseg):
            acc = jnp.zeros((nlane,), jnp.float32)
            for t in range(8):
                seg = bins[2][t, pl.ds(c * nlane, nlane)]
                acc = jnp.where(idxvec == t, seg, acc)
            bins[k][0, pl.ds(c * nlane, nlane)] = acc

    pending_in = {}
    pending_out = {}
    pending_in[0] = start_in(0)
    pending_in[1] = start_in(1)
    for i in range(_NCH):
        k = i % 3
        pending_in[i].wait()
        if i == 0:
            @pl.when(q == 0)
            def _(k=k):
                shuffle_desc(k)
                patch_token(k)

            @pl.when(q > 0)
            def _(k=k):
                shuffle_asc(k)
        else:
            shuffle_asc(k)
        pending_out[i] = start_out(i)
        if i + 2 < _NCH:
            if i - 1 >= 0:
                pending_out[i - 1].wait()
            pending_in[i + 2] = start_in(i + 2)
    for i in (_NCH - 3, _NCH - 2, _NCH - 1):
        pending_out[i].wait()

    @pl.when(q == 7)
    def _():
        klast = (_NCH - 1) % 3
        shuf_row(klast, 0, _W - 1)
        pltpu.sync_copy(bins[klast].at[pl.ds(0, 1)],
                        out_hbm.at[b, pl.ds(S, 1), :])


def kernel(x, modality_id, tokens):
    B, S, D = x.shape
    idx = jnp.full((16,), modality_id, jnp.int32)
    mesh = plsc.VectorSubcoreMesh(core_axis_name="c", subcore_axis_name="s")
    k = pl.kernel(
        functools.partial(_sc_body, B=B, S=S, D=D),
        out_type=jax.ShapeDtypeStruct((B, S + 1, D), x.dtype),
        mesh=mesh,
        compiler_params=pltpu.CompilerParams(use_tc_tiling_on_sc=True),
        scratch_types=[
            pltpu.VMEM((_W, D), x.dtype),
            pltpu.VMEM((_W, D), x.dtype),
            pltpu.VMEM((_W, D), x.dtype),
            pltpu.VMEM((16,), jnp.int32),
            pltpu.SemaphoreType.DMA,
            pltpu.SemaphoreType.DMA,
            pltpu.SemaphoreType.DMA,
            pltpu.SemaphoreType.DMA,
            pltpu.SemaphoreType.DMA,
            pltpu.SemaphoreType.DMA,
        ],
    )
    return k(x, idx, tokens)


# speedup vs baseline: 2.9043x; 1.3476x over previous
import functools

import jax
import jax.numpy as jnp
from jax import lax
from jax.experimental import pallas as pl
from jax.experimental.pallas import tpu as pltpu
from jax.experimental.pallas import tpu_sc as plsc

_CH = 32
_H = 16
_W = _CH + 8
_NCH = 16
_LANE = 16


def _sc_body(x_hbm, idx_hbm, tok_hbm, out_hbm,
             bin0, bin1, bout0, bout1, idx_v,
             si0, si1, so0, so1, *, B, S, D):
    nc = 2
    bins = (bin0, bin1)
    bouts = (bout0, bout1)
    sin = (si0, si1)
    sout = (so0, so1)
    nseg = D // _LANE

    wid = lax.axis_index("s") * nc + lax.axis_index("c")
    q = wid % 8
    b = wid // 8
    r0 = q * (_CH * _NCH)

    pltpu.sync_copy(idx_hbm, idx_v)
    idxvec = idx_v[...]

    @pl.when(q == 0)
    def _():
        pltpu.sync_copy(tok_hbm, bouts[1].at[pl.ds(0, 8)])

    def start_in(i):
        k = i % 2
        ws = pl.multiple_of(jnp.maximum(r0 + i * _CH - 8, 0), 8)
        return pltpu.async_copy(
            x_hbm.at[b, pl.ds(ws, _W), :], bins[k], sin[k])

    def start_out(i, h):
        os_ = pl.multiple_of(r0 + i * _CH + h * _H, 8)
        return pltpu.async_copy(
            bouts[h], out_hbm.at[b, pl.ds(os_, _H), :], sout[h])

    def shuffle(i, h, shift):
        kb = i % 2
        dst = bouts[h]
        src = bins[kb]

        def body(r, _):
            sr = jnp.maximum(h * _H + r + shift, 0)
            for cb in range(4):
                vals = [src[sr, pl.ds((cb * 16 + c) * _LANE, _LANE)]
                        for c in range(16)]
                for c in range(16):
                    dst[r, pl.ds((cb * 16 + c) * _LANE, _LANE)] = vals[c]
            return 0

        lax.fori_loop(0, _H, body, 0)

    def patch_token():
        for c in range(nseg):
            acc = jnp.zeros((_LANE,), jnp.float32)
            for t in range(8):
                seg = bouts[1][t, pl.ds(c * _LANE, _LANE)]
                acc = jnp.where(idxvec == t, seg, acc)
            bouts[0][0, pl.ds(c * _LANE, _LANE)] = acc

    pending_in = {}
    pending_out = {}
    pending_in[0] = start_in(0)
    pending_in[1] = start_in(1)
    for i in range(_NCH):
        pending_in[i].wait()
        shift = jnp.where(jnp.logical_and(q == 0, i == 0), -1, 7) if i == 0 else 7
        for h in (0, 1):
            j = 2 * i + h
            if j - 2 >= 0:
                pending_out[j - 2].wait()
            shuffle(i, h, shift)
            if i == 0 and h == 0:
                @pl.when(q == 0)
                def _():
                    patch_token()
            pending_out[j] = start_out(i, h)
        if i + 2 < _NCH:
            pending_in[i + 2] = start_in(i + 2)
    pending_out[2 * _NCH - 2].wait()
    pending_out[2 * _NCH - 1].wait()

    @pl.when(q == 7)
    def _():
        klast = (_NCH - 1) % 2
        for c in range(nseg):
            bouts[0][0, pl.ds(c * _LANE, _LANE)] = (
                bins[klast][_W - 1, pl.ds(c * _LANE, _LANE)])
        pltpu.sync_copy(bouts[0].at[pl.ds(0, 1)],
                        out_hbm.at[b, pl.ds(S, 1), :])


def kernel(x, modality_id, tokens):
    B, S, D = x.shape
    idx = jnp.full((16,), modality_id, jnp.int32)
    mesh = plsc.VectorSubcoreMesh(core_axis_name="c", subcore_axis_name="s")
    k = pl.kernel(
        functools.partial(_sc_body, B=B, S=S, D=D),
        out_type=jax.ShapeDtypeStruct((B, S + 1, D), x.dtype),
        mesh=mesh,
        compiler_params=pltpu.CompilerParams(use_tc_tiling_on_sc=True),
        scratch_types=[
            pltpu.VMEM((_W, D), x.dtype),
            pltpu.VMEM((_W, D), x.dtype),
            pltpu.VMEM((_H, D), x.dtype),
            pltpu.VMEM((_H, D), x.dtype),
            pltpu.VMEM((16,), jnp.int32),
            pltpu.SemaphoreType.DMA,
            pltpu.SemaphoreType.DMA,
            pltpu.SemaphoreType.DMA,
            pltpu.SemaphoreType.DMA,
        ],
    )
    return k(x, idx, tokens)


# speedup vs baseline: 3.1092x; 1.0705x over previous
import functools

import jax
import jax.numpy as jnp
from jax import lax
from jax.experimental import pallas as pl
from jax.experimental.pallas import tpu as pltpu
from jax.experimental.pallas import tpu_sc as plsc

_CH = 32
_NCH = 16
_LANE = 16


def _sc_body(x_hbm, idx_hbm, tok_hbm, out_hbm,
             b0, b1, b2, i0, i1, i2, idxt_v,
             sg0, sg1, sg2, so0, so1, so2, st, *, B, S, D):
    nc = 2
    bins = (b0, b1, b2)
    idxs = (i0, i1, i2)
    sg = (sg0, sg1, sg2)
    so = (so0, so1, so2)
    nseg = D // _LANE

    wid = lax.axis_index("s") * nc + lax.axis_index("c")
    q = wid % 8
    b = wid // 8
    r0 = q * (_CH * _NCH)

    pltpu.sync_copy(idx_hbm, idxt_v)

    def fill_idx(i):
        k = i % 3
        base = b * S + r0 + i * _CH - 1
        lanes = lax.iota(jnp.int32, _LANE)
        idxs[k][pl.ds(0, _LANE)] = jnp.maximum(base + lanes, 0)
        idxs[k][pl.ds(_LANE, _LANE)] = base + _LANE + lanes

    def start_gather(i):
        k = i % 3
        fill_idx(i)
        return pltpu.async_copy(x_hbm.at[idxs[k]], bins[k], sg[k])

    def start_out(i):
        k = i % 3
        os_ = pl.multiple_of(r0 + i * _CH, 8)
        return pltpu.async_copy(
            bins[k], out_hbm.at[b, pl.ds(os_, _CH), :], so[k])

    pending_g = {}
    pending_o = {}
    pending_g[0] = start_gather(0)
    pending_g[1] = start_gather(1)
    for i in range(_NCH):
        k = i % 3
        pending_g[i].wait()
        if i == 0:
            @pl.when(q == 0)
            def _():
                pltpu.async_copy(
                    tok_hbm.at[idxt_v], bins[2].at[pl.ds(0, _LANE)], st
                ).wait()
                for c in range(nseg):
                    bins[0][0, pl.ds(c * _LANE, _LANE)] = (
                        bins[2][0, pl.ds(c * _LANE, _LANE)])
        pending_o[i] = start_out(i)
        if i + 2 < _NCH:
            if i - 1 >= 0:
                pending_o[i - 1].wait()
            pending_g[i + 2] = start_gather(i + 2)
    for i in (_NCH - 3, _NCH - 2, _NCH - 1):
        pending_o[i].wait()

    @pl.when(q == 7)
    def _():
        klast = (_NCH - 1) % 3
        idxt_v[...] = jnp.full((_LANE,), b * S + S - 1, jnp.int32)
        pltpu.async_copy(
            x_hbm.at[idxt_v], bins[klast].at[pl.ds(0, _LANE)], st).wait()
        pltpu.sync_copy(bins[klast].at[pl.ds(0, 1)],
                        out_hbm.at[b, pl.ds(S, 1), :])


def kernel(x, modality_id, tokens):
    B, S, D = x.shape
    x2 = x.reshape(B * S, D)
    idx = jnp.full((16,), modality_id, jnp.int32)
    mesh = plsc.VectorSubcoreMesh(core_axis_name="c", subcore_axis_name="s")
    k = pl.kernel(
        functools.partial(_sc_body, B=B, S=S, D=D),
        out_type=jax.ShapeDtypeStruct((B, S + 1, D), x.dtype),
        mesh=mesh,
        compiler_params=pltpu.CompilerParams(use_tc_tiling_on_sc=True),
        scratch_types=[
            pltpu.VMEM((_CH, D), x.dtype),
            pltpu.VMEM((_CH, D), x.dtype),
            pltpu.VMEM((_CH, D), x.dtype),
            pltpu.VMEM((_CH,), jnp.int32),
            pltpu.VMEM((_CH,), jnp.int32),
            pltpu.VMEM((_CH,), jnp.int32),
            pltpu.VMEM((_LANE,), jnp.int32),
            pltpu.SemaphoreType.DMA,
            pltpu.SemaphoreType.DMA,
            pltpu.SemaphoreType.DMA,
            pltpu.SemaphoreType.DMA,
            pltpu.SemaphoreType.DMA,
            pltpu.SemaphoreType.DMA,
            pltpu.SemaphoreType.DMA,
        ],
    )
    return k(x2, idx, tokens)


# speedup vs baseline: 4.1587x; 1.3375x over previous
import functools

import jax
import jax.numpy as jnp
from jax import lax
from jax.experimental import pallas as pl
from jax.experimental.pallas import tpu as pltpu
from jax.experimental.pallas import tpu_sc as plsc

_CH = 32
_NCH = 16
_LANE = 16


def _sc_body(x_hbm, idx_hbm, tok_hbm, out_hbm,
             b0, b1, b2, i0, i1, i2, idxt_v,
             sg0, sg1, sg2, so0, so1, so2, st, *, B, S, D):
    nc = 2
    bins = (b0, b1, b2)
    idxs = (i0, i1, i2)
    sg = (sg0, sg1, sg2)
    so = (so0, so1, so2)
    nseg = D // _LANE

    wid = lax.axis_index("s") * nc + lax.axis_index("c")
    q = wid % 8
    b = wid // 8
    r0 = q * (_CH * _NCH)

    pltpu.sync_copy(idx_hbm, idxt_v)

    def fill_idx(i):
        k = i % 3
        base = b * S + r0 + i * _CH - 1
        lanes = lax.iota(jnp.int32, _LANE)
        idxs[k][pl.ds(0, _LANE)] = jnp.maximum(base + lanes, 0)
        idxs[k][pl.ds(_LANE, _LANE)] = base + _LANE + lanes

    def start_gather(i):
        k = i % 3
        fill_idx(i)
        return pltpu.async_copy(x_hbm.at[idxs[k]], bins[k], sg[k])

    def start_out(i):
        k = i % 3
        os_ = pl.multiple_of(r0 + i * _CH, 8)
        return pltpu.async_copy(
            bins[k], out_hbm.at[b, pl.ds(os_, _CH), :], so[k])

    pending_g = {}
    pending_o = {}
    pending_g[0] = start_gather(0)
    pending_g[1] = start_gather(1)
    for i in range(_NCH):
        k = i % 3
        pending_g[i].wait()
        if i == 0:
            @pl.when(q == 0)
            def _():
                pltpu.async_copy(
                    tok_hbm.at[idxt_v], bins[2].at[pl.ds(0, _LANE)], st
                ).wait()
                for c in range(nseg):
                    bins[0][0, pl.ds(c * _LANE, _LANE)] = (
                        bins[2][0, pl.ds(c * _LANE, _LANE)])
        pending_o[i] = start_out(i)
        if i + 2 < _NCH:
            if i - 1 >= 0:
                pending_o[i - 1].wait()
            pending_g[i + 2] = start_gather(i + 2)
    for i in (_NCH - 3, _NCH - 2, _NCH - 1):
        pending_o[i].wait()

    @pl.when(q == 7)
    def _():
        klast = (_NCH - 1) % 3
        idxt_v[...] = jnp.full((_LANE,), b * S + S - 1, jnp.int32)
        pltpu.async_copy(
            x_hbm.at[idxt_v], bins[klast].at[pl.ds(0, _LANE)], st).wait()
        pltpu.sync_copy(bins[klast].at[pl.ds(0, 1)],
                        out_hbm.at[b, pl.ds(S, 1), :])


def kernel(x, modality_id, tokens):
    B, S, D = x.shape
    x2 = x.reshape(B * S, D)
    idx = jnp.full((16,), modality_id, jnp.int32)
    mesh = plsc.VectorSubcoreMesh(core_axis_name="c", subcore_axis_name="s")
    k = pl.kernel(
        functools.partial(_sc_body, B=B, S=S, D=D),
        out_type=jax.ShapeDtypeStruct((B, S + 8, D), x.dtype),
        mesh=mesh,
        compiler_params=pltpu.CompilerParams(use_tc_tiling_on_sc=True),
        scratch_types=[
            pltpu.VMEM((_CH, D), x.dtype),
            pltpu.VMEM((_CH, D), x.dtype),
            pltpu.VMEM((_CH, D), x.dtype),
            pltpu.VMEM((_CH,), jnp.int32),
            pltpu.VMEM((_CH,), jnp.int32),
            pltpu.VMEM((_CH,), jnp.int32),
            pltpu.VMEM((_LANE,), jnp.int32),
            pltpu.SemaphoreType.DMA,
            pltpu.SemaphoreType.DMA,
            pltpu.SemaphoreType.DMA,
            pltpu.SemaphoreType.DMA,
            pltpu.SemaphoreType.DMA,
            pltpu.SemaphoreType.DMA,
            pltpu.SemaphoreType.DMA,
        ],
    )
    return k(x2, idx, tokens)[:, : S + 1, :]


# speedup vs baseline: 6.5561x; 1.5765x over previous
import functools

import jax
import jax.numpy as jnp
from jax import lax
from jax.experimental import pallas as pl
from jax.experimental.pallas import tpu as pltpu
from jax.experimental.pallas import tpu_sc as plsc

_CH = 8
_HP = 4
_NCH = 16
_LANE = 16


def _sc_body(x_hbm, idx_hbm, tok_hbm, out_hbm,
             g0, g1, o0, o1, tokbuf, gi0, gi1, ti,
             sg0, sg1, so0, so1, st, *, B, S, D):
    nc = 2
    gbufs = (g0, g1)
    obufs = (o0, o1)
    gidx = (gi0, gi1)
    sg = (sg0, sg1)
    so = (so0, so1)
    nseg = D // 128
    kdim = B * nseg

    wid = lax.axis_index("s") * nc + lax.axis_index("c")
    p_base = wid * (_CH * _NCH)

    @pl.when(wid == 0)
    def _():
        pltpu.sync_copy(idx_hbm, ti)
        pltpu.async_copy(tok_hbm.at[ti], tokbuf, st).wait()

    def fill_idx(i):
        k = i % 2
        p0 = p_base + i * _CH
        for half in range(2):
            j = lax.iota(jnp.int32, _LANE) + half * _LANE
            b = jnp.bitwise_and(j, B - 1)
            r = jnp.right_shift(j, 2)
            gidx[k][pl.ds(half * _LANE, _LANE)] = jnp.maximum(
                b * S + p0 + r - 1, 0)

    def start_gather(i):
        k = i % 2
        fill_idx(i)
        return pltpu.async_copy(x_hbm.at[gidx[k]], gbufs[k], sg[k])

    def start_out(i, h):
        p0 = p_base + i * _CH + h * _HP
        return pltpu.async_copy(
            obufs[h], out_hbm.at[pl.ds(p0, _HP)], so[h])

    def interleave(i, h):
        src = gbufs[i % 2]
        dst = obufs[h]

        def body(pc, _):
            p = pc // nseg
            c = pc % nseg
            for b in range(B):
                vals = [src[(h * _HP + p) * B + b,
                            pl.ds(c * 128 + v * _LANE, _LANE)]
                        for v in range(8)]
                for v in range(8):
                    dst[p, c * B + b, pl.ds(v * _LANE, _LANE)] = vals[v]
            return 0

        lax.fori_loop(0, _HP * nseg, body, 0)

    def patch_token():
        for c in range(nseg):
            for v in range(8):
                seg = tokbuf[0, pl.ds(c * 128 + v * _LANE, _LANE)]
                for b in range(B):
                    obufs[0][0, c * B + b, pl.ds(v * _LANE, _LANE)] = seg

    pending_g = {}
    pending_o = {}
    pending_g[0] = start_gather(0)
    pending_g[1] = start_gather(1)
    for i in range(_NCH):
        pending_g[i].wait()
        for h in (0, 1):
            j = 2 * i + h
            if j - 2 >= 0:
                pending_o[j - 2].wait()
            interleave(i, h)
            if i == 0 and h == 0:
                @pl.when(wid == 0)
                def _():
                    patch_token()
            pending_o[j] = start_out(i, h)
        if i + 2 < _NCH:
            pending_g[i + 2] = start_gather(i + 2)
    pending_o[2 * _NCH - 2].wait()
    pending_o[2 * _NCH - 1].wait()

    @pl.when(wid == 31)
    def _():
        j = lax.iota(jnp.int32, _LANE)
        gidx[0][pl.ds(0, _LANE)] = jnp.bitwise_and(j, B - 1) * S + (S - 1)
        gidx[0][pl.ds(_LANE, _LANE)] = (
            jnp.bitwise_and(j, B - 1) * S + (S - 1))
        pltpu.async_copy(x_hbm.at[gidx[0]], gbufs[0], sg[0]).wait()
        for c in range(nseg):
            for b in range(B):
                for v in range(8):
                    obufs[0][0, c * B + b, pl.ds(v * _LANE, _LANE)] = (
                        gbufs[0][b, pl.ds(c * 128 + v * _LANE, _LANE)])
        pltpu.sync_copy(obufs[0].at[pl.ds(0, 1)],
                        out_hbm.at[pl.ds(S, 1)])


def kernel(x, modality_id, tokens):
    B, S, D = x.shape
    nseg = D // 128
    x2 = x.reshape(B * S, D)
    idx = jnp.full((16,), modality_id, jnp.int32)
    mesh = plsc.VectorSubcoreMesh(core_axis_name="c", subcore_axis_name="s")
    k = pl.kernel(
        functools.partial(_sc_body, B=B, S=S, D=D),
        out_type=jax.ShapeDtypeStruct((S + 1, B * nseg, 128), x.dtype),
        mesh=mesh,
        compiler_params=pltpu.CompilerParams(use_tc_tiling_on_sc=True),
        scratch_types=[
            pltpu.VMEM((_CH * B, D), x.dtype),
            pltpu.VMEM((_CH * B, D), x.dtype),
            pltpu.VMEM((_HP, B * nseg, 128), x.dtype),
            pltpu.VMEM((_HP, B * nseg, 128), x.dtype),
            pltpu.VMEM((16, D), x.dtype),
            pltpu.VMEM((_CH * B,), jnp.int32),
            pltpu.VMEM((_CH * B,), jnp.int32),
            pltpu.VMEM((16,), jnp.int32),
            pltpu.SemaphoreType.DMA,
            pltpu.SemaphoreType.DMA,
            pltpu.SemaphoreType.DMA,
            pltpu.SemaphoreType.DMA,
            pltpu.SemaphoreType.DMA,
        ],
    )
    p3 = k(x2, idx, tokens)
    return (p3.reshape(S + 1, nseg, B, 128)
            .transpose(2, 0, 1, 3)
            .reshape(B, S + 1, D))
